# baseline (speedup 1.0000x reference)
import functools

import jax
import jax.numpy as jnp
from jax import lax
from jax.experimental import pallas as pl
from jax.experimental.pallas import tpu as pltpu
from jax.experimental.pallas import tpu_sc as plsc

N = 10000
E = 160000
IN = 256
HID = 512
HEADS = 16
DH = HID // HEADS
L = 3
OUT = 128

NPAD = 10240
SLICE = NPAD // 16
EPT = E // 16
CHUNK = 128
NCH = (EPT + CHUNK - 1) // CHUNK
EPT_P = NCH * CHUNK
NV = CHUNK // 16
NV_REAL = EPT // 16
HPC = HEADS // 2

BM = 400
GRID = N // BM

_f32 = jnp.float32


def _a0_body(x_ref, w_ref, b_ref, o_ref):
    o_ref[...] = (
        jnp.dot(x_ref[...], w_ref[...], preferred_element_type=_f32)
        + b_ref[...]
    )


def _tc_input_proj(x, W, b):
    return pl.pallas_call(
        _a0_body,
        grid=(GRID,),
        in_specs=[
            pl.BlockSpec((BM, IN), lambda i: (i, 0)),
            pl.BlockSpec((IN, HID), lambda i: (0, 0)),
            pl.BlockSpec((1, HID), lambda i: (0, 0)),
        ],
        out_specs=pl.BlockSpec((BM, HID), lambda i: (i, 0)),
        out_shape=jax.ShapeDtypeStruct((N, HID), _f32),
    )(x, W, b)


def _make_layer_body(pre):
    def body(*refs):
        if pre:
            (u_ref, s_ref, b_ref, w_ref, al_ref, ar_ref,
             feat_ref, el_ref, er_ref, mx_ref) = refs
        else:
            (u_ref, w_ref, al_ref, ar_ref,
             feat_ref, el_ref, er_ref, mx_ref) = refs
        i = pl.program_id(0)
        a = u_ref[...]
        if pre:
            a = jnp.maximum(a / (s_ref[...] + 1e-9) + b_ref[...], 0.0)
        f = jnp.dot(a, w_ref[...], preferred_element_type=_f32)
        feat_ref[...] = f
        el = jnp.dot(f, al_ref[...], preferred_element_type=_f32)
        er = jnp.dot(f, ar_ref[...], preferred_element_type=_f32)
        el_ref[...] = el
        er_ref[...] = er
        elm = jnp.max(el, axis=0, keepdims=True)
        erm = jnp.max(er, axis=0, keepdims=True)
        new = jnp.concatenate(
            [elm, erm, jnp.full((6, HEADS), -jnp.inf, _f32)], axis=0)

        @pl.when(i == 0)
        def _():
            mx_ref[...] = new

        @pl.when(i > 0)
        def _():
            mx_ref[...] = jnp.maximum(mx_ref[...], new)

    return body


_layer_body_pre = _make_layer_body(True)
_layer_body_nopre = _make_layer_body(False)

_LAYER_OUT = (
    jax.ShapeDtypeStruct((N, HID), _f32),
    jax.ShapeDtypeStruct((N, HEADS), _f32),
    jax.ShapeDtypeStruct((N, HEADS), _f32),
    jax.ShapeDtypeStruct((8, HEADS), _f32),
)

_LAYER_OUT_SPECS = [
    pl.BlockSpec((BM, HID), lambda i: (i, 0)),
    pl.BlockSpec((BM, HEADS), lambda i: (i, 0)),
    pl.BlockSpec((BM, HEADS), lambda i: (i, 0)),
    pl.BlockSpec((8, HEADS), lambda i: (0, 0)),
]


def _tc_layer_first(h, W, albd, arbd):
    return pl.pallas_call(
        _layer_body_nopre,
        grid=(GRID,),
        in_specs=[
            pl.BlockSpec((BM, HID), lambda i: (i, 0)),
            pl.BlockSpec((HID, HID), lambda i: (0, 0)),
            pl.BlockSpec((HID, HEADS), lambda i: (0, 0)),
            pl.BlockSpec((HID, HEADS), lambda i: (0, 0)),
        ],
        out_specs=_LAYER_OUT_SPECS,
        out_shape=_LAYER_OUT,
    )(h, W, albd, arbd)


def _tc_layer_next(u, esr, b, W, albd, arbd):
    return pl.pallas_call(
        _layer_body_pre,
        grid=(GRID,),
        in_specs=[
            pl.BlockSpec((BM, HID), lambda i: (i, 0)),
            pl.BlockSpec((BM, HID), lambda i: (i, 0)),
            pl.BlockSpec((1, HID), lambda i: (0, 0)),
            pl.BlockSpec((HID, HID), lambda i: (0, 0)),
            pl.BlockSpec((HID, HEADS), lambda i: (0, 0)),
            pl.BlockSpec((HID, HEADS), lambda i: (0, 0)),
        ],
        out_specs=_LAYER_OUT_SPECS,
        out_shape=_LAYER_OUT,
    )(u, esr, b, W, albd, arbd)


def _pool_body(u_ref, s_ref, b_ref, wk_ref, bk_ref, wv_ref, bv_ref, q_ref,
               w1_ref, b1_ref, w2_ref, b2_ref, o_ref, acc, sm):
    i = pl.program_id(0)
    a = jnp.maximum(u_ref[...] / (s_ref[...] + 1e-9) + b_ref[...], 0.0)
    kk = jnp.dot(a, wk_ref[...], preferred_element_type=_f32) + bk_ref[...]
    vv = jnp.dot(a, wv_ref[...], preferred_element_type=_f32) + bv_ref[...]
    lg = jnp.sum(kk * q_ref[...], axis=1, keepdims=True) * _f32(HID ** -0.5)
    bm = jnp.max(lg)

    @pl.when(i == 0)
    def _():
        sm[0] = -jnp.inf
        sm[1] = 0.0
        acc[...] = jnp.zeros((8, HID), _f32)

    prev_m = sm[0]
    prev_s = sm[1]
    prev_v = acc[0:1, :]
    new_m = jnp.maximum(prev_m, bm)
    corr = jnp.exp(prev_m - new_m)
    p = jnp.exp(lg - new_m)
    sm[0] = new_m
    sm[1] = prev_s * corr + jnp.sum(p)
    acc[0:1, :] = prev_v * corr + jnp.sum(p * vv, axis=0, keepdims=True)

    @pl.when(i == GRID - 1)
    def _():
        hg = acc[0:1, :] / sm[1]
        o1 = jnp.maximum(
            jnp.dot(hg, w1_ref[...], preferred_element_type=_f32)
            + b1_ref[...], 0.0)
        o_ref[...] = (
            jnp.dot(o1, w2_ref[...], preferred_element_type=_f32)
            + b2_ref[...]
        )


def _tc_pool(u, esr, b, Wk, bk, Wv, bv, q, W1, b1, W2, b2):
    return pl.pallas_call(
        _pool_body,
        grid=(GRID,),
        in_specs=[
            pl.BlockSpec((BM, HID), lambda i: (i, 0)),
            pl.BlockSpec((BM, HID), lambda i: (i, 0)),
            pl.BlockSpec((1, HID), lambda i: (0, 0)),
            pl.BlockSpec((HID, HID), lambda i: (0, 0)),
            pl.BlockSpec((1, HID), lambda i: (0, 0)),
            pl.BlockSpec((HID, HID), lambda i: (0, 0)),
            pl.BlockSpec((1, HID), lambda i: (0, 0)),
            pl.BlockSpec((1, HID), lambda i: (0, 0)),
            pl.BlockSpec((HID, HID), lambda i: (0, 0)),
            pl.BlockSpec((1, HID), lambda i: (0, 0)),
            pl.BlockSpec((HID, OUT), lambda i: (0, 0)),
            pl.BlockSpec((1, OUT), lambda i: (0, 0)),
        ],
        out_specs=pl.BlockSpec((1, OUT), lambda i: (0, 0)),
        out_shape=jax.ShapeDtypeStruct((1, OUT), _f32),
        scratch_shapes=[
            pltpu.VMEM((8, HID), _f32),
            pltpu.SMEM((2,), _f32),
        ],
    )(u, esr, b, Wk, bk, Wv, bv, q, W1, b1, W2, b2)


_mesh = plsc.VectorSubcoreMesh(
    core_axis_name="c", subcore_axis_name="s", num_cores=2, num_subcores=16)


@functools.partial(
    pl.kernel,
    out_type=(
        jax.ShapeDtypeStruct((HEADS, NPAD, DH), _f32),
        jax.ShapeDtypeStruct((HEADS, NPAD), _f32),
    ),
    mesh=_mesh,
    compiler_params=pltpu.CompilerParams(
        use_tc_tiling_on_sc=False, needs_layout_passes=False),
    scratch_types=[
        pltpu.VMEM((N,), _f32),
        pltpu.VMEM((N,), _f32),
        pltpu.VMEM((16,), _f32),
        pltpu.VMEM((NCH, CHUNK), jnp.int32),
        pltpu.VMEM((NCH, CHUNK), jnp.int32),
        pltpu.VMEM((NCH + 4, CHUNK), jnp.int32),
        pltpu.VMEM((NCH, CHUNK), _f32),
        pltpu.VMEM((4, CHUNK, DH), _f32),
        pltpu.VMEM((CHUNK, DH), _f32),
        pltpu.VMEM((CHUNK, DH), _f32),
        pltpu.VMEM((SLICE,), _f32),
        pltpu.VMEM_SHARED((NPAD,), _f32),
        pltpu.VMEM_SHARED((NPAD, DH), _f32),
        pltpu.SemaphoreType.DMA,
    ],
)
def _sc_layer(featv, elT, erT, cb, srcp, dstp, out_u, esumT,
              el_v, er_v, cvec, src_v, dst_v, gix_v, ee_v, gbuf, sbuf,
              zb_v, zs_v, esum_s, out_s, gsem):
    c = lax.axis_index("c")
    s = lax.axis_index("s")
    pltpu.sync_copy(srcp.at[s], src_v)
    pltpu.sync_copy(dstp.at[s], dst_v)

    zero = jnp.zeros((16,), _f32)
    izero = jnp.zeros((16,), jnp.int32)
    for t in range(4):
        for k in range(NV):
            gix_v[NCH + t, pl.ds(k * 16, 16)] = izero

    def zb_loop(r, carry):
        zb_v[r, pl.ds(0, 16)] = zero
        zb_v[r, pl.ds(16, 16)] = zero
        return carry

    lax.fori_loop(0, CHUNK, zb_loop, 0)

    def zs_loop(r, carry):
        zs_v[pl.ds(r * 16, 16)] = zero
        return carry

    lax.fori_loop(0, SLICE // 16, zs_loop, 0)

    def head_body(hl, carry):
        h = c * HPC + hl
        pltpu.sync_copy(elT.at[h], el_v)
        pltpu.sync_copy(erT.at[h], er_v)
        pltpu.sync_copy(cb.at[h], cvec)
        pltpu.sync_copy(zs_v, esum_s.at[pl.ds(s * SLICE, SLICE)])
        for kk in range(SLICE // CHUNK):
            pltpu.sync_copy(
                zb_v, out_s.at[pl.ds(s * SLICE + kk * CHUNK, CHUNK)])
        plsc.subcore_barrier()

        cv = cvec[...]

        def p1(j, carry):
            for k in range(NV):
                sl = pl.ds(k * 16, 16)
                sv = src_v[j, sl]
                dv = dst_v[j, sl]
                av = plsc.load_gather(el_v, [sv])
                bv2 = plsc.load_gather(er_v, [dv])
                e = av + bv2
                e = jnp.where(e > 0, e, e * 0.2)
                ee = jnp.exp(e - cv)
                ee = jnp.where(j * NV + k < NV_REAL, ee, jnp.zeros_like(ee))
                ee_v[j, sl] = ee
                gix_v[j, sl] = sv * HEADS + h
            pltpu.sync_copy(ee_v.at[j], esum_s.at[dst_v.at[j]], add=True)
            return carry

        lax.fori_loop(0, NCH, p1, 0)
        plsc.subcore_barrier()

        for b in range(4):
            pltpu.async_copy(featv.at[gix_v.at[b]], gbuf.at[b], gsem)

        def p2(j, carry):
            b = j & 3
            pltpu.make_async_copy(
                featv.at[pl.ds(0, CHUNK)], gbuf.at[0], gsem).wait()
            for k in range(NV):
                ee = ee_v[j, pl.ds(k * 16, 16)]
                for i2 in range(16):
                    r = k * 16 + i2
                    asp = jnp.broadcast_to(ee[i2], (16,))
                    sbuf[r, pl.ds(0, 16)] = gbuf[b, r, pl.ds(0, 16)] * asp
                    sbuf[r, pl.ds(16, 16)] = gbuf[b, r, pl.ds(16, 16)] * asp
            pltpu.async_copy(featv.at[gix_v.at[j + 4]], gbuf.at[b], gsem)
            pltpu.sync_copy(sbuf, out_s.at[dst_v.at[j]], add=True)
            return carry

        lax.fori_loop(0, NCH, p2, 0)
        for b in range(4):
            pltpu.make_async_copy(
                featv.at[pl.ds(0, CHUNK)], gbuf.at[0], gsem).wait()
        plsc.subcore_barrier()
        pltpu.sync_copy(out_s.at[pl.ds(s * SLICE, SLICE)],
                        out_u.at[h, pl.ds(s * SLICE, SLICE)])
        pltpu.sync_copy(esum_s.at[pl.ds(s * SLICE, SLICE)],
                        esumT.at[h, pl.ds(s * SLICE, SLICE)])
        plsc.subcore_barrier()
        return carry

    lax.fori_loop(0, HPC, head_body, 0)


def kernel(x, edge_index, W_in, b_in, Wl, al, ar, bl, q, Wk, bk, Wv, bv,
           W1, b1, W2, b2):
    src = edge_index[0]
    dst = edge_index[1]
    srcp = jnp.pad(src.reshape(16, EPT),
                   ((0, 0), (0, EPT_P - EPT))).reshape(16, NCH, CHUNK)
    dstp = jnp.pad(dst.reshape(16, EPT),
                   ((0, 0), (0, EPT_P - EPT))).reshape(16, NCH, CHUNK)

    h = _tc_input_proj(x, W_in, b_in.reshape(1, HID))

    karr = jnp.arange(HID)
    hsel = (karr[:, None] // DH) == jnp.arange(HEADS)[None, :]

    u = None
    esr = None
    for i in range(L):
        albd = jnp.where(hsel, al[i].reshape(HID, 1), 0.0).astype(_f32)
        arbd = jnp.where(hsel, ar[i].reshape(HID, 1), 0.0).astype(_f32)
        if i == 0:
            feat, el, er, mx = _tc_layer_first(h, Wl[i], albd, arbd)
        else:
            feat, el, er, mx = _tc_layer_next(
                u, esr, bl[i - 1].reshape(1, HID), Wl[i], albd, arbd)
        cmax = jnp.maximum(0.0, mx[0] + mx[1])
        cb = jnp.broadcast_to(cmax[:, None], (HEADS, 16)).astype(_f32)
        featv = feat.reshape(N * HEADS, DH)
        elT = el.T
        erT = er.T
        out_u, esumT = _sc_layer(featv, elT, erT, cb, srcp, dstp)
        u = out_u[:, :N, :].transpose(1, 0, 2).reshape(N, HID)
        esr = jnp.repeat(esumT[:, :N].T, DH, axis=1)

    return _tc_pool(u, esr, bl[L - 1].reshape(1, HID), Wk,
                    bk.reshape(1, HID), Wv, bv.reshape(1, HID), q,
                    W1, b1.reshape(1, HID), W2, b2.reshape(1, OUT))


# speedup vs baseline: 1.3412x; 1.3412x over previous
import functools

import jax
import jax.numpy as jnp
from jax import lax
from jax.experimental import pallas as pl
from jax.experimental.pallas import tpu as pltpu
from jax.experimental.pallas import tpu_sc as plsc

N = 10000
E = 160000
IN = 256
HID = 512
HEADS = 16
DH = HID // HEADS
L = 3
OUT = 128

NPAD = 10240
SLICE = NPAD // 16
EPT = E // 16
CHUNK = 128
NCH = (EPT + CHUNK - 1) // CHUNK
EPT_P = NCH * CHUNK
NV = CHUNK // 16
NV_REAL = EPT // 16
HPC = HEADS // 2

BM = 400
GRID = N // BM

_f32 = jnp.float32


def _a0_body(x_ref, w_ref, b_ref, o_ref):
    o_ref[...] = (
        jnp.dot(x_ref[...], w_ref[...], preferred_element_type=_f32)
        + b_ref[...]
    )


def _tc_input_proj(x, W, b):
    return pl.pallas_call(
        _a0_body,
        grid=(GRID,),
        in_specs=[
            pl.BlockSpec((BM, IN), lambda i: (i, 0)),
            pl.BlockSpec((IN, HID), lambda i: (0, 0)),
            pl.BlockSpec((1, HID), lambda i: (0, 0)),
        ],
        out_specs=pl.BlockSpec((BM, HID), lambda i: (i, 0)),
        out_shape=jax.ShapeDtypeStruct((N, HID), _f32),
    )(x, W, b)


def _make_layer_body(pre):
    def body(*refs):
        if pre:
            (u_ref, s_ref, b_ref, w_ref, al_ref, ar_ref,
             feat_ref, el_ref, er_ref, mx_ref) = refs
        else:
            (u_ref, w_ref, al_ref, ar_ref,
             feat_ref, el_ref, er_ref, mx_ref) = refs
        i = pl.program_id(0)
        a = u_ref[...]
        if pre:
            a = jnp.maximum(a / (s_ref[...] + 1e-9) + b_ref[...], 0.0)
        f = jnp.dot(a, w_ref[...], preferred_element_type=_f32)
        feat_ref[...] = f
        el = jnp.dot(f, al_ref[...], preferred_element_type=_f32)
        er = jnp.dot(f, ar_ref[...], preferred_element_type=_f32)
        el_ref[...] = el
        er_ref[...] = er
        elm = jnp.max(el, axis=0, keepdims=True)
        erm = jnp.max(er, axis=0, keepdims=True)
        new = jnp.concatenate(
            [elm, erm, jnp.full((6, HEADS), -jnp.inf, _f32)], axis=0)

        @pl.when(i == 0)
        def _():
            mx_ref[...] = new

        @pl.when(i > 0)
        def _():
            mx_ref[...] = jnp.maximum(mx_ref[...], new)

    return body


_layer_body_pre = _make_layer_body(True)
_layer_body_nopre = _make_layer_body(False)

_LAYER_OUT = (
    jax.ShapeDtypeStruct((N, HID), _f32),
    jax.ShapeDtypeStruct((N, HEADS), _f32),
    jax.ShapeDtypeStruct((N, HEADS), _f32),
    jax.ShapeDtypeStruct((8, HEADS), _f32),
)

_LAYER_OUT_SPECS = [
    pl.BlockSpec((BM, HID), lambda i: (i, 0)),
    pl.BlockSpec((BM, HEADS), lambda i: (i, 0)),
    pl.BlockSpec((BM, HEADS), lambda i: (i, 0)),
    pl.BlockSpec((8, HEADS), lambda i: (0, 0)),
]


def _tc_layer_first(h, W, albd, arbd):
    return pl.pallas_call(
        _layer_body_nopre,
        grid=(GRID,),
        in_specs=[
            pl.BlockSpec((BM, HID), lambda i: (i, 0)),
            pl.BlockSpec((HID, HID), lambda i: (0, 0)),
            pl.BlockSpec((HID, HEADS), lambda i: (0, 0)),
            pl.BlockSpec((HID, HEADS), lambda i: (0, 0)),
        ],
        out_specs=_LAYER_OUT_SPECS,
        out_shape=_LAYER_OUT,
    )(h, W, albd, arbd)


def _tc_layer_next(u, esr, b, W, albd, arbd):
    return pl.pallas_call(
        _layer_body_pre,
        grid=(GRID,),
        in_specs=[
            pl.BlockSpec((BM, HID), lambda i: (i, 0)),
            pl.BlockSpec((BM, HID), lambda i: (i, 0)),
            pl.BlockSpec((1, HID), lambda i: (0, 0)),
            pl.BlockSpec((HID, HID), lambda i: (0, 0)),
            pl.BlockSpec((HID, HEADS), lambda i: (0, 0)),
            pl.BlockSpec((HID, HEADS), lambda i: (0, 0)),
        ],
        out_specs=_LAYER_OUT_SPECS,
        out_shape=_LAYER_OUT,
    )(u, esr, b, W, albd, arbd)


def _pool_body(u_ref, s_ref, b_ref, wk_ref, bk_ref, wv_ref, bv_ref, q_ref,
               w1_ref, b1_ref, w2_ref, b2_ref, o_ref, acc, sm):
    i = pl.program_id(0)
    a = jnp.maximum(u_ref[...] / (s_ref[...] + 1e-9) + b_ref[...], 0.0)
    kk = jnp.dot(a, wk_ref[...], preferred_element_type=_f32) + bk_ref[...]
    vv = jnp.dot(a, wv_ref[...], preferred_element_type=_f32) + bv_ref[...]
    lg = jnp.sum(kk * q_ref[...], axis=1, keepdims=True) * _f32(HID ** -0.5)
    bm = jnp.max(lg)

    @pl.when(i == 0)
    def _():
        sm[0] = -jnp.inf
        sm[1] = 0.0
        acc[...] = jnp.zeros((8, HID), _f32)

    prev_m = sm[0]
    prev_s = sm[1]
    prev_v = acc[0:1, :]
    new_m = jnp.maximum(prev_m, bm)
    corr = jnp.exp(prev_m - new_m)
    p = jnp.exp(lg - new_m)
    sm[0] = new_m
    sm[1] = prev_s * corr + jnp.sum(p)
    acc[0:1, :] = prev_v * corr + jnp.sum(p * vv, axis=0, keepdims=True)

    @pl.when(i == GRID - 1)
    def _():
        hg = acc[0:1, :] / sm[1]
        o1 = jnp.maximum(
            jnp.dot(hg, w1_ref[...], preferred_element_type=_f32)
            + b1_ref[...], 0.0)
        o_ref[...] = (
            jnp.dot(o1, w2_ref[...], preferred_element_type=_f32)
            + b2_ref[...]
        )


def _tc_pool(u, esr, b, Wk, bk, Wv, bv, q, W1, b1, W2, b2):
    return pl.pallas_call(
        _pool_body,
        grid=(GRID,),
        in_specs=[
            pl.BlockSpec((BM, HID), lambda i: (i, 0)),
            pl.BlockSpec((BM, HID), lambda i: (i, 0)),
            pl.BlockSpec((1, HID), lambda i: (0, 0)),
            pl.BlockSpec((HID, HID), lambda i: (0, 0)),
            pl.BlockSpec((1, HID), lambda i: (0, 0)),
            pl.BlockSpec((HID, HID), lambda i: (0, 0)),
            pl.BlockSpec((1, HID), lambda i: (0, 0)),
            pl.BlockSpec((1, HID), lambda i: (0, 0)),
            pl.BlockSpec((HID, HID), lambda i: (0, 0)),
            pl.BlockSpec((1, HID), lambda i: (0, 0)),
            pl.BlockSpec((HID, OUT), lambda i: (0, 0)),
            pl.BlockSpec((1, OUT), lambda i: (0, 0)),
        ],
        out_specs=pl.BlockSpec((1, OUT), lambda i: (0, 0)),
        out_shape=jax.ShapeDtypeStruct((1, OUT), _f32),
        scratch_shapes=[
            pltpu.VMEM((8, HID), _f32),
            pltpu.SMEM((2,), _f32),
        ],
    )(u, esr, b, Wk, bk, Wv, bv, q, W1, b1, W2, b2)


_mesh = plsc.VectorSubcoreMesh(
    core_axis_name="c", subcore_axis_name="s", num_cores=2, num_subcores=16)


@functools.partial(
    pl.kernel,
    out_type=(
        jax.ShapeDtypeStruct((HEADS, NPAD, DH), _f32),
        jax.ShapeDtypeStruct((HEADS, NPAD), _f32),
    ),
    mesh=_mesh,
    compiler_params=pltpu.CompilerParams(
        use_tc_tiling_on_sc=False, needs_layout_passes=False),
    scratch_types=[
        pltpu.VMEM((N,), _f32),
        pltpu.VMEM((N,), _f32),
        pltpu.VMEM((16,), _f32),
        pltpu.VMEM((NCH, CHUNK), jnp.int32),
        pltpu.VMEM((NCH, CHUNK), jnp.int32),
        pltpu.VMEM((NCH + 4, CHUNK), jnp.int32),
        pltpu.VMEM((NCH, CHUNK), _f32),
        pltpu.VMEM((4, CHUNK, DH), _f32),
        pltpu.VMEM((CHUNK, DH), _f32),
        pltpu.VMEM((CHUNK, DH), _f32),
        pltpu.VMEM((SLICE,), _f32),
        pltpu.VMEM_SHARED((NPAD,), _f32),
        pltpu.VMEM_SHARED((NPAD, DH), _f32),
        pltpu.SemaphoreType.DMA,
    ],
)
def _sc_layer(featv, elT, erT, cb, srcp, dstp, out_u, esumT,
              el_v, er_v, cvec, src_v, dst_v, gix_v, ee_v, gbuf, sbuf,
              zb_v, zs_v, esum_s, out_s, gsem):
    c = lax.axis_index("c")
    s = lax.axis_index("s")
    pltpu.sync_copy(srcp.at[s], src_v)
    pltpu.sync_copy(dstp.at[s], dst_v)

    zero = jnp.zeros((16,), _f32)
    izero = jnp.zeros((16,), jnp.int32)
    for t in range(4):
        for k in range(NV):
            gix_v[NCH + t, pl.ds(k * 16, 16)] = izero

    def zb_loop(r, carry):
        zb_v[r, pl.ds(0, 16)] = zero
        zb_v[r, pl.ds(16, 16)] = zero
        return carry

    lax.fori_loop(0, CHUNK, zb_loop, 0)

    def zs_loop(r, carry):
        zs_v[pl.ds(r * 16, 16)] = zero
        return carry

    lax.fori_loop(0, SLICE // 16, zs_loop, 0)

    def head_body(hl, carry):
        h = c * HPC + hl
        pltpu.sync_copy(elT.at[h], el_v)
        pltpu.sync_copy(erT.at[h], er_v)
        pltpu.sync_copy(cb.at[h], cvec)
        pltpu.sync_copy(zs_v, esum_s.at[pl.ds(s * SLICE, SLICE)])
        for kk in range(SLICE // CHUNK):
            pltpu.sync_copy(
                zb_v, out_s.at[pl.ds(s * SLICE + kk * CHUNK, CHUNK)])
        plsc.subcore_barrier()

        cv = cvec[...]

        def p1(j, carry):
            for k in range(NV):
                sl = pl.ds(k * 16, 16)
                sv = src_v[j, sl]
                dv = dst_v[j, sl]
                av = plsc.load_gather(el_v, [sv])
                bv2 = plsc.load_gather(er_v, [dv])
                e = av + bv2
                e = jnp.where(e > 0, e, e * 0.2)
                ee = jnp.exp(e - cv)
                ee = jnp.where(j * NV + k < NV_REAL, ee, jnp.zeros_like(ee))
                ee_v[j, sl] = ee
                gix_v[j, sl] = sv * HEADS + h
            pltpu.sync_copy(ee_v.at[j], esum_s.at[dst_v.at[j]], add=True)
            return carry

        lax.fori_loop(0, NCH, p1, 0)
        plsc.subcore_barrier()

        for b in range(4):
            pltpu.async_copy(featv.at[gix_v.at[b]], gbuf.at[b], gsem)

        def _p2_step(j, b):
            pltpu.make_async_copy(
                featv.at[pl.ds(0, CHUNK)], gbuf.at[0], gsem).wait()
            for k in range(NV):
                ee = ee_v[j, pl.ds(k * 16, 16)]
                for i2 in range(16):
                    r = k * 16 + i2
                    asp = jnp.broadcast_to(ee[i2], (16,))
                    sbuf[r, pl.ds(0, 16)] = gbuf[b, r, pl.ds(0, 16)] * asp
                    sbuf[r, pl.ds(16, 16)] = gbuf[b, r, pl.ds(16, 16)] * asp
            pltpu.async_copy(featv.at[gix_v.at[j + 4]], gbuf.at[b], gsem)
            pltpu.sync_copy(sbuf, out_s.at[dst_v.at[j]], add=True)

        def p2group(p, carry):
            for b in range(4):
                _p2_step(p * 4 + b, b)
            return carry

        lax.fori_loop(0, NCH // 4, p2group, 0)
        for j in range(NCH // 4 * 4, NCH):
            _p2_step(j, j % 4)
        for b in range(4):
            pltpu.make_async_copy(
                featv.at[pl.ds(0, CHUNK)], gbuf.at[0], gsem).wait()
        plsc.subcore_barrier()
        pltpu.sync_copy(out_s.at[pl.ds(s * SLICE, SLICE)],
                        out_u.at[h, pl.ds(s * SLICE, SLICE)])
        pltpu.sync_copy(esum_s.at[pl.ds(s * SLICE, SLICE)],
                        esumT.at[h, pl.ds(s * SLICE, SLICE)])
        plsc.subcore_barrier()
        return carry

    lax.fori_loop(0, HPC, head_body, 0)


def kernel(x, edge_index, W_in, b_in, Wl, al, ar, bl, q, Wk, bk, Wv, bv,
           W1, b1, W2, b2):
    src = edge_index[0]
    dst = edge_index[1]
    srcp = jnp.pad(src.reshape(16, EPT),
                   ((0, 0), (0, EPT_P - EPT))).reshape(16, NCH, CHUNK)
    dstp = jnp.pad(dst.reshape(16, EPT),
                   ((0, 0), (0, EPT_P - EPT))).reshape(16, NCH, CHUNK)

    h = _tc_input_proj(x, W_in, b_in.reshape(1, HID))

    karr = jnp.arange(HID)
    hsel = (karr[:, None] // DH) == jnp.arange(HEADS)[None, :]

    u = None
    esr = None
    for i in range(L):
        albd = jnp.where(hsel, al[i].reshape(HID, 1), 0.0).astype(_f32)
        arbd = jnp.where(hsel, ar[i].reshape(HID, 1), 0.0).astype(_f32)
        if i == 0:
            feat, el, er, mx = _tc_layer_first(h, Wl[i], albd, arbd)
        else:
            feat, el, er, mx = _tc_layer_next(
                u, esr, bl[i - 1].reshape(1, HID), Wl[i], albd, arbd)
        cmax = jnp.maximum(0.0, mx[0] + mx[1])
        cb = jnp.broadcast_to(cmax[:, None], (HEADS, 16)).astype(_f32)
        featv = feat.reshape(N * HEADS, DH)
        elT = el.T
        erT = er.T
        out_u, esumT = _sc_layer(featv, elT, erT, cb, srcp, dstp)
        u = out_u[:, :N, :].transpose(1, 0, 2).reshape(N, HID)
        esr = jnp.repeat(esumT[:, :N].T, DH, axis=1)

    return _tc_pool(u, esr, bl[L - 1].reshape(1, HID), Wk,
                    bk.reshape(1, HID), Wv, bv.reshape(1, HID), q,
                    W1, b1.reshape(1, HID), W2, b2.reshape(1, OUT))


# speedup vs baseline: 2.5555x; 1.9054x over previous
import functools

import jax
import jax.numpy as jnp
from jax import lax
from jax.experimental import pallas as pl
from jax.experimental.pallas import tpu as pltpu
from jax.experimental.pallas import tpu_sc as plsc

N = 10000
E = 160000
IN = 256
HID = 512
HEADS = 16
DH = HID // HEADS
L = 3
OUT = 128

NPAD = 10240
SLICE = NPAD // 16
EPT = E // 16
CHUNK = 128
NCH = (EPT + CHUNK - 1) // CHUNK
EPT_P = NCH * CHUNK
NV = CHUNK // 16
NV_REAL = EPT // 16
HPC = HEADS // 2

BM = 400
GRID = N // BM

_f32 = jnp.float32


def _a0_body(x_ref, w_ref, b_ref, o_ref):
    o_ref[...] = (
        jnp.dot(x_ref[...], w_ref[...], preferred_element_type=_f32)
        + b_ref[...]
    )


def _tc_input_proj(x, W, b):
    return pl.pallas_call(
        _a0_body,
        grid=(GRID,),
        in_specs=[
            pl.BlockSpec((BM, IN), lambda i: (i, 0)),
            pl.BlockSpec((IN, HID), lambda i: (0, 0)),
            pl.BlockSpec((1, HID), lambda i: (0, 0)),
        ],
        out_specs=pl.BlockSpec((BM, HID), lambda i: (i, 0)),
        out_shape=jax.ShapeDtypeStruct((N, HID), _f32),
    )(x, W, b)


def _make_layer_body(pre):
    def body(*refs):
        if pre:
            (u_ref, s_ref, b_ref, w_ref, al_ref, ar_ref,
             feat_ref, el_ref, er_ref, mx_ref) = refs
        else:
            (u_ref, w_ref, al_ref, ar_ref,
             feat_ref, el_ref, er_ref, mx_ref) = refs
        i = pl.program_id(0)
        a = u_ref[...]
        if pre:
            a = jnp.maximum(a / (s_ref[...] + 1e-9) + b_ref[...], 0.0)
        f = jnp.dot(a, w_ref[...], preferred_element_type=_f32)
        feat_ref[...] = f
        el = jnp.dot(f, al_ref[...], preferred_element_type=_f32)
        er = jnp.dot(f, ar_ref[...], preferred_element_type=_f32)
        el_ref[...] = el
        er_ref[...] = er
        elm = jnp.max(el, axis=0, keepdims=True)
        erm = jnp.max(er, axis=0, keepdims=True)
        new = jnp.concatenate(
            [elm, erm, jnp.full((6, HEADS), -jnp.inf, _f32)], axis=0)

        @pl.when(i == 0)
        def _():
            mx_ref[...] = new

        @pl.when(i > 0)
        def _():
            mx_ref[...] = jnp.maximum(mx_ref[...], new)

    return body


_layer_body_pre = _make_layer_body(True)
_layer_body_nopre = _make_layer_body(False)

_LAYER_OUT = (
    jax.ShapeDtypeStruct((N, HID), _f32),
    jax.ShapeDtypeStruct((N, HEADS), _f32),
    jax.ShapeDtypeStruct((N, HEADS), _f32),
    jax.ShapeDtypeStruct((8, HEADS), _f32),
)

_LAYER_OUT_SPECS = [
    pl.BlockSpec((BM, HID), lambda i: (i, 0)),
    pl.BlockSpec((BM, HEADS), lambda i: (i, 0)),
    pl.BlockSpec((BM, HEADS), lambda i: (i, 0)),
    pl.BlockSpec((8, HEADS), lambda i: (0, 0)),
]


def _tc_layer_first(h, W, albd, arbd):
    return pl.pallas_call(
        _layer_body_nopre,
        grid=(GRID,),
        in_specs=[
            pl.BlockSpec((BM, HID), lambda i: (i, 0)),
            pl.BlockSpec((HID, HID), lambda i: (0, 0)),
            pl.BlockSpec((HID, HEADS), lambda i: (0, 0)),
            pl.BlockSpec((HID, HEADS), lambda i: (0, 0)),
        ],
        out_specs=_LAYER_OUT_SPECS,
        out_shape=_LAYER_OUT,
    )(h, W, albd, arbd)


def _tc_layer_next(u, esr, b, W, albd, arbd):
    return pl.pallas_call(
        _layer_body_pre,
        grid=(GRID,),
        in_specs=[
            pl.BlockSpec((BM, HID), lambda i: (i, 0)),
            pl.BlockSpec((BM, HID), lambda i: (i, 0)),
            pl.BlockSpec((1, HID), lambda i: (0, 0)),
            pl.BlockSpec((HID, HID), lambda i: (0, 0)),
            pl.BlockSpec((HID, HEADS), lambda i: (0, 0)),
            pl.BlockSpec((HID, HEADS), lambda i: (0, 0)),
        ],
        out_specs=_LAYER_OUT_SPECS,
        out_shape=_LAYER_OUT,
    )(u, esr, b, W, albd, arbd)


def _pool_body(u_ref, s_ref, b_ref, wk_ref, bk_ref, wv_ref, bv_ref, q_ref,
               w1_ref, b1_ref, w2_ref, b2_ref, o_ref, acc, sm):
    i = pl.program_id(0)
    a = jnp.maximum(u_ref[...] / (s_ref[...] + 1e-9) + b_ref[...], 0.0)
    kk = jnp.dot(a, wk_ref[...], preferred_element_type=_f32) + bk_ref[...]
    vv = jnp.dot(a, wv_ref[...], preferred_element_type=_f32) + bv_ref[...]
    lg = jnp.sum(kk * q_ref[...], axis=1, keepdims=True) * _f32(HID ** -0.5)
    bm = jnp.max(lg)

    @pl.when(i == 0)
    def _():
        sm[0] = -jnp.inf
        sm[1] = 0.0
        acc[...] = jnp.zeros((8, HID), _f32)

    prev_m = sm[0]
    prev_s = sm[1]
    prev_v = acc[0:1, :]
    new_m = jnp.maximum(prev_m, bm)
    corr = jnp.exp(prev_m - new_m)
    p = jnp.exp(lg - new_m)
    sm[0] = new_m
    sm[1] = prev_s * corr + jnp.sum(p)
    acc[0:1, :] = prev_v * corr + jnp.sum(p * vv, axis=0, keepdims=True)

    @pl.when(i == GRID - 1)
    def _():
        hg = acc[0:1, :] / sm[1]
        o1 = jnp.maximum(
            jnp.dot(hg, w1_ref[...], preferred_element_type=_f32)
            + b1_ref[...], 0.0)
        o_ref[...] = (
            jnp.dot(o1, w2_ref[...], preferred_element_type=_f32)
            + b2_ref[...]
        )


def _tc_pool(u, esr, b, Wk, bk, Wv, bv, q, W1, b1, W2, b2):
    return pl.pallas_call(
        _pool_body,
        grid=(GRID,),
        in_specs=[
            pl.BlockSpec((BM, HID), lambda i: (i, 0)),
            pl.BlockSpec((BM, HID), lambda i: (i, 0)),
            pl.BlockSpec((1, HID), lambda i: (0, 0)),
            pl.BlockSpec((HID, HID), lambda i: (0, 0)),
            pl.BlockSpec((1, HID), lambda i: (0, 0)),
            pl.BlockSpec((HID, HID), lambda i: (0, 0)),
            pl.BlockSpec((1, HID), lambda i: (0, 0)),
            pl.BlockSpec((1, HID), lambda i: (0, 0)),
            pl.BlockSpec((HID, HID), lambda i: (0, 0)),
            pl.BlockSpec((1, HID), lambda i: (0, 0)),
            pl.BlockSpec((HID, OUT), lambda i: (0, 0)),
            pl.BlockSpec((1, OUT), lambda i: (0, 0)),
        ],
        out_specs=pl.BlockSpec((1, OUT), lambda i: (0, 0)),
        out_shape=jax.ShapeDtypeStruct((1, OUT), _f32),
        scratch_shapes=[
            pltpu.VMEM((8, HID), _f32),
            pltpu.SMEM((2,), _f32),
        ],
    )(u, esr, b, Wk, bk, Wv, bv, q, W1, b1, W2, b2)


_mesh = plsc.VectorSubcoreMesh(
    core_axis_name="c", subcore_axis_name="s", num_cores=2, num_subcores=16)


@functools.partial(
    pl.kernel,
    out_type=(
        jax.ShapeDtypeStruct((HEADS, NPAD, DH), _f32),
        jax.ShapeDtypeStruct((HEADS, NPAD), _f32),
    ),
    mesh=_mesh,
    compiler_params=pltpu.CompilerParams(
        use_tc_tiling_on_sc=False, needs_layout_passes=False),
    scratch_types=[
        pltpu.VMEM((N,), _f32),
        pltpu.VMEM((N,), _f32),
        pltpu.VMEM((16,), _f32),
        pltpu.VMEM((NCH, CHUNK), jnp.int32),
        pltpu.VMEM((NCH, CHUNK), jnp.int32),
        pltpu.VMEM((NCH, CHUNK), jnp.int32),
        pltpu.VMEM((NCH, CHUNK), _f32),
        pltpu.VMEM((2, CHUNK, DH), _f32),
        pltpu.VMEM((CHUNK, DH), _f32),
        pltpu.VMEM((CHUNK, DH), _f32),
        pltpu.VMEM((SLICE,), _f32),
        pltpu.VMEM_SHARED((NPAD,), _f32),
        pltpu.VMEM_SHARED((NPAD, DH), _f32),
        pltpu.SemaphoreType.DMA,
        pltpu.SemaphoreType.DMA,
    ],
)
def _sc_layer(featv, elT, erT, cb, srcp, dstp, out_u, esumT,
              el_v, er_v, cvec, src_v, dst_v, gix_v, ee_v, gbuf, sbuf,
              zb_v, zs_v, esum_s, out_s, gsem, gsem2):
    c = lax.axis_index("c")
    s = lax.axis_index("s")
    pltpu.sync_copy(srcp.at[s], src_v)
    pltpu.sync_copy(dstp.at[s], dst_v)

    zero = jnp.zeros((16,), _f32)

    def zb_loop(r, carry):
        zb_v[r, pl.ds(0, 16)] = zero
        zb_v[r, pl.ds(16, 16)] = zero
        return carry

    lax.fori_loop(0, CHUNK, zb_loop, 0)

    def zs_loop(r, carry):
        zs_v[pl.ds(r * 16, 16)] = zero
        return carry

    lax.fori_loop(0, SLICE // 16, zs_loop, 0)

    def head_body(hl, carry):
        h = c * HPC + hl
        pltpu.sync_copy(elT.at[h], el_v)
        pltpu.sync_copy(erT.at[h], er_v)
        pltpu.sync_copy(cb.at[h], cvec)
        pltpu.sync_copy(zs_v, esum_s.at[pl.ds(s * SLICE, SLICE)])
        for kk in range(SLICE // CHUNK):
            pltpu.sync_copy(
                zb_v, out_s.at[pl.ds(s * SLICE + kk * CHUNK, CHUNK)])
        plsc.subcore_barrier()

        cv = cvec[...]

        def p1(j, carry):
            for k in range(NV):
                sl = pl.ds(k * 16, 16)
                sv = src_v[j, sl]
                dv = dst_v[j, sl]
                av = plsc.load_gather(el_v, [sv])
                bv2 = plsc.load_gather(er_v, [dv])
                e = av + bv2
                e = jnp.where(e > 0, e, e * 0.2)
                ee = jnp.exp(e - cv)
                ee = jnp.where(j * NV + k < NV_REAL, ee, jnp.zeros_like(ee))
                ee_v[j, sl] = ee
                gix_v[j, sl] = sv * HEADS + h
            pltpu.sync_copy(ee_v.at[j], esum_s.at[dst_v.at[j]], add=True)
            return carry

        lax.fori_loop(0, NCH, p1, 0)
        plsc.subcore_barrier()

        def _p2_work(j, b):
            for k in range(NV):
                ee = ee_v[j, pl.ds(k * 16, 16)]
                for i2 in range(16):
                    r = k * 16 + i2
                    asp = jnp.broadcast_to(ee[i2], (16,))
                    sbuf[r, pl.ds(0, 16)] = gbuf[b, r, pl.ds(0, 16)] * asp
                    sbuf[r, pl.ds(16, 16)] = gbuf[b, r, pl.ds(16, 16)] * asp
            pltpu.sync_copy(sbuf, out_s.at[dst_v.at[j]], add=True)

        def _g(j, b, sem):
            return pltpu.make_async_copy(featv.at[gix_v.at[j]],
                                         gbuf.at[b], sem)

        pltpu.async_copy(featv.at[gix_v.at[0]], gbuf.at[0], gsem)

        def p2pair(p, carry):
            j0 = p * 2
            _g(j0 + 1, 1, gsem2).start()
            _g(j0, 0, gsem).wait()
            _p2_work(j0, 0)
            _g(j0 + 2, 0, gsem).start()
            _g(j0 + 1, 1, gsem2).wait()
            _p2_work(j0 + 1, 1)
            return carry

        lax.fori_loop(0, NCH // 2, p2pair, 0)
        _g(NCH - 1, 0, gsem).wait()
        _p2_work(NCH - 1, 0)
        plsc.subcore_barrier()
        pltpu.sync_copy(out_s.at[pl.ds(s * SLICE, SLICE)],
                        out_u.at[h, pl.ds(s * SLICE, SLICE)])
        pltpu.sync_copy(esum_s.at[pl.ds(s * SLICE, SLICE)],
                        esumT.at[h, pl.ds(s * SLICE, SLICE)])
        plsc.subcore_barrier()
        return carry

    lax.fori_loop(0, HPC, head_body, 0)


def kernel(x, edge_index, W_in, b_in, Wl, al, ar, bl, q, Wk, bk, Wv, bv,
           W1, b1, W2, b2):
    src = edge_index[0]
    dst = edge_index[1]
    srcp = jnp.pad(src.reshape(16, EPT),
                   ((0, 0), (0, EPT_P - EPT))).reshape(16, NCH, CHUNK)
    dstp = jnp.pad(dst.reshape(16, EPT),
                   ((0, 0), (0, EPT_P - EPT))).reshape(16, NCH, CHUNK)

    h = _tc_input_proj(x, W_in, b_in.reshape(1, HID))

    karr = jnp.arange(HID)
    hsel = (karr[:, None] // DH) == jnp.arange(HEADS)[None, :]

    u = None
    esr = None
    for i in range(L):
        albd = jnp.where(hsel, al[i].reshape(HID, 1), 0.0).astype(_f32)
        arbd = jnp.where(hsel, ar[i].reshape(HID, 1), 0.0).astype(_f32)
        if i == 0:
            feat, el, er, mx = _tc_layer_first(h, Wl[i], albd, arbd)
        else:
            feat, el, er, mx = _tc_layer_next(
                u, esr, bl[i - 1].reshape(1, HID), Wl[i], albd, arbd)
        cmax = jnp.maximum(0.0, mx[0] + mx[1])
        cb = jnp.broadcast_to(cmax[:, None], (HEADS, 16)).astype(_f32)
        featv = feat.reshape(N * HEADS, DH)
        elT = el.T
        erT = er.T
        out_u, esumT = _sc_layer(featv, elT, erT, cb, srcp, dstp)
        u = out_u[:, :N, :].transpose(1, 0, 2).reshape(N, HID)
        esr = jnp.repeat(esumT[:, :N].T, DH, axis=1)

    return _tc_pool(u, esr, bl[L - 1].reshape(1, HID), Wk,
                    bk.reshape(1, HID), Wv, bv.reshape(1, HID), q,
                    W1, b1.reshape(1, HID), W2, b2.reshape(1, OUT))


# speedup vs baseline: 2.5712x; 1.0061x over previous
import functools

import jax
import jax.numpy as jnp
from jax import lax
from jax.experimental import pallas as pl
from jax.experimental.pallas import tpu as pltpu
from jax.experimental.pallas import tpu_sc as plsc

N = 10000
E = 160000
IN = 256
HID = 512
HEADS = 16
DH = HID // HEADS
L = 3
OUT = 128

NPAD = 10240
SLICE = NPAD // 16
EPT = E // 16
CHUNK = 128
NCH = (EPT + CHUNK - 1) // CHUNK
EPT_P = NCH * CHUNK
NV = CHUNK // 16
NV_REAL = EPT // 16
HPC = HEADS // 2

BM = 512
GRID = NPAD // BM

_f32 = jnp.float32


def _a0_body(x_ref, w_ref, b_ref, o_ref):
    o_ref[...] = (
        jnp.dot(x_ref[...], w_ref[...], preferred_element_type=_f32)
        + b_ref[...]
    )


def _tc_input_proj(x, W, b):
    return pl.pallas_call(
        _a0_body,
        grid=(GRID,),
        in_specs=[
            pl.BlockSpec((BM, IN), lambda i: (i, 0)),
            pl.BlockSpec((IN, HID), lambda i: (0, 0)),
            pl.BlockSpec((1, HID), lambda i: (0, 0)),
        ],
        out_specs=pl.BlockSpec((BM, HID), lambda i: (i, 0)),
        out_shape=jax.ShapeDtypeStruct((NPAD, HID), _f32),
    )(x, W, b)


def _make_layer_body(pre):
    def body(*refs):
        if pre:
            (u_ref, b_ref, w_ref, al_ref, ar_ref,
             feat_ref, el_ref, er_ref, mx_ref) = refs
        else:
            (u_ref, w_ref, al_ref, ar_ref,
             feat_ref, el_ref, er_ref, mx_ref) = refs
        i = pl.program_id(0)
        a = u_ref[...]
        if pre:
            a = jnp.maximum(a + b_ref[...], 0.0)
        f = jnp.dot(a, w_ref[...], preferred_element_type=_f32)
        feat_ref[...] = f
        el = jnp.dot(f, al_ref[...], preferred_element_type=_f32)
        er = jnp.dot(f, ar_ref[...], preferred_element_type=_f32)
        el_ref[...] = el.T
        er_ref[...] = er.T
        elm = jnp.max(el, axis=0, keepdims=True)
        erm = jnp.max(er, axis=0, keepdims=True)
        new = jnp.concatenate(
            [elm, erm, jnp.full((6, HEADS), -jnp.inf, _f32)], axis=0)

        @pl.when(i == 0)
        def _():
            mx_ref[...] = new

        @pl.when(i > 0)
        def _():
            mx_ref[...] = jnp.maximum(mx_ref[...], new)

    return body


_layer_body_pre = _make_layer_body(True)
_layer_body_nopre = _make_layer_body(False)

_LAYER_OUT = (
    jax.ShapeDtypeStruct((NPAD, HID), _f32),
    jax.ShapeDtypeStruct((HEADS, NPAD), _f32),
    jax.ShapeDtypeStruct((HEADS, NPAD), _f32),
    jax.ShapeDtypeStruct((8, HEADS), _f32),
)

_LAYER_OUT_SPECS = [
    pl.BlockSpec((BM, HID), lambda i: (i, 0)),
    pl.BlockSpec((HEADS, BM), lambda i: (0, i)),
    pl.BlockSpec((HEADS, BM), lambda i: (0, i)),
    pl.BlockSpec((8, HEADS), lambda i: (0, 0)),
]


def _tc_layer_first(h, W, albd, arbd):
    return pl.pallas_call(
        _layer_body_nopre,
        grid=(GRID,),
        in_specs=[
            pl.BlockSpec((BM, HID), lambda i: (i, 0)),
            pl.BlockSpec((HID, HID), lambda i: (0, 0)),
            pl.BlockSpec((HID, HEADS), lambda i: (0, 0)),
            pl.BlockSpec((HID, HEADS), lambda i: (0, 0)),
        ],
        out_specs=_LAYER_OUT_SPECS,
        out_shape=_LAYER_OUT,
    )(h, W, albd, arbd)


def _tc_layer_next(u, b, W, albd, arbd):
    return pl.pallas_call(
        _layer_body_pre,
        grid=(GRID,),
        in_specs=[
            pl.BlockSpec((BM, HID), lambda i: (i, 0)),
            pl.BlockSpec((1, HID), lambda i: (0, 0)),
            pl.BlockSpec((HID, HID), lambda i: (0, 0)),
            pl.BlockSpec((HID, HEADS), lambda i: (0, 0)),
            pl.BlockSpec((HID, HEADS), lambda i: (0, 0)),
        ],
        out_specs=_LAYER_OUT_SPECS,
        out_shape=_LAYER_OUT,
    )(u, b, W, albd, arbd)


def _pool_body(u_ref, b_ref, wk_ref, bk_ref, wv_ref, bv_ref, q_ref,
               w1_ref, b1_ref, w2_ref, b2_ref, o_ref, acc, sm):
    i = pl.program_id(0)
    a = jnp.maximum(u_ref[...] + b_ref[...], 0.0)
    kk = jnp.dot(a, wk_ref[...], preferred_element_type=_f32) + bk_ref[...]
    vv = jnp.dot(a, wv_ref[...], preferred_element_type=_f32) + bv_ref[...]
    lg = jnp.sum(kk * q_ref[...], axis=1, keepdims=True) * _f32(HID ** -0.5)
    rows = i * BM + jax.lax.broadcasted_iota(jnp.int32, (BM, 1), 0)
    lg = jnp.where(rows < N, lg, -jnp.inf)
    bm = jnp.max(lg)

    @pl.when(i == 0)
    def _():
        sm[0] = -jnp.inf
        sm[1] = 0.0
        acc[...] = jnp.zeros((8, HID), _f32)

    prev_m = sm[0]
    prev_s = sm[1]
    prev_v = acc[0:1, :]
    new_m = jnp.maximum(prev_m, bm)
    corr = jnp.exp(prev_m - new_m)
    p = jnp.exp(lg - new_m)
    sm[0] = new_m
    sm[1] = prev_s * corr + jnp.sum(p)
    acc[0:1, :] = prev_v * corr + jnp.sum(p * vv, axis=0, keepdims=True)

    @pl.when(i == GRID - 1)
    def _():
        hg = acc[0:1, :] / sm[1]
        o1 = jnp.maximum(
            jnp.dot(hg, w1_ref[...], preferred_element_type=_f32)
            + b1_ref[...], 0.0)
        o_ref[...] = (
            jnp.dot(o1, w2_ref[...], preferred_element_type=_f32)
            + b2_ref[...]
        )


def _tc_pool(u, b, Wk, bk, Wv, bv, q, W1, b1, W2, b2):
    return pl.pallas_call(
        _pool_body,
        grid=(GRID,),
        in_specs=[
            pl.BlockSpec((BM, HID), lambda i: (i, 0)),
            pl.BlockSpec((1, HID), lambda i: (0, 0)),
            pl.BlockSpec((HID, HID), lambda i: (0, 0)),
            pl.BlockSpec((1, HID), lambda i: (0, 0)),
            pl.BlockSpec((HID, HID), lambda i: (0, 0)),
            pl.BlockSpec((1, HID), lambda i: (0, 0)),
            pl.BlockSpec((1, HID), lambda i: (0, 0)),
            pl.BlockSpec((HID, HID), lambda i: (0, 0)),
            pl.BlockSpec((1, HID), lambda i: (0, 0)),
            pl.BlockSpec((HID, OUT), lambda i: (0, 0)),
            pl.BlockSpec((1, OUT), lambda i: (0, 0)),
        ],
        out_specs=pl.BlockSpec((1, OUT), lambda i: (0, 0)),
        out_shape=jax.ShapeDtypeStruct((1, OUT), _f32),
        scratch_shapes=[
            pltpu.VMEM((8, HID), _f32),
            pltpu.SMEM((2,), _f32),
        ],
    )(u, b, Wk, bk, Wv, bv, q, W1, b1, W2, b2)


_mesh = plsc.VectorSubcoreMesh(
    core_axis_name="c", subcore_axis_name="s", num_cores=2, num_subcores=16)


@functools.partial(
    pl.kernel,
    out_type=jax.ShapeDtypeStruct((NPAD, HEADS, DH), _f32),
    mesh=_mesh,
    compiler_params=pltpu.CompilerParams(
        use_tc_tiling_on_sc=False, needs_layout_passes=False),
    scratch_types=[
        pltpu.VMEM((NPAD,), _f32),
        pltpu.VMEM((NPAD,), _f32),
        pltpu.VMEM((16,), _f32),
        pltpu.VMEM((NCH, CHUNK), jnp.int32),
        pltpu.VMEM((NCH, CHUNK), jnp.int32),
        pltpu.VMEM((NCH, CHUNK), jnp.int32),
        pltpu.VMEM((NCH, CHUNK), _f32),
        pltpu.VMEM((2, CHUNK, DH), _f32),
        pltpu.VMEM((CHUNK, DH), _f32),
        pltpu.VMEM((CHUNK, DH), _f32),
        pltpu.VMEM((SLICE,), _f32),
        pltpu.VMEM((SLICE,), _f32),
        pltpu.VMEM_SHARED((NPAD,), _f32),
        pltpu.VMEM_SHARED((NPAD, DH), _f32),
        pltpu.SemaphoreType.DMA,
        pltpu.SemaphoreType.DMA,
    ],
)
def _sc_layer(featv, elT, erT, cb, srcp, dstp, out_u,
              el_v, er_v, cvec, src_v, dst_v, gix_v, ee_v, gbuf, sbuf,
              zb_v, zs_v, es_v, esum_s, out_s, gsem, gsem2):
    c = lax.axis_index("c")
    s = lax.axis_index("s")
    pltpu.sync_copy(srcp.at[s], src_v)
    pltpu.sync_copy(dstp.at[s], dst_v)

    zero = jnp.zeros((16,), _f32)

    def zb_loop(r, carry):
        zb_v[r, pl.ds(0, 16)] = zero
        zb_v[r, pl.ds(16, 16)] = zero
        return carry

    lax.fori_loop(0, CHUNK, zb_loop, 0)

    def zs_loop(r, carry):
        zs_v[pl.ds(r * 16, 16)] = zero
        return carry

    lax.fori_loop(0, SLICE // 16, zs_loop, 0)

    def head_body(hl, carry):
        h = c * HPC + hl
        pltpu.sync_copy(elT.at[h], el_v)
        pltpu.sync_copy(erT.at[h], er_v)
        pltpu.sync_copy(cb.at[h], cvec)
        pltpu.sync_copy(zs_v, esum_s.at[pl.ds(s * SLICE, SLICE)])
        for kk in range(SLICE // CHUNK):
            pltpu.sync_copy(
                zb_v, out_s.at[pl.ds(s * SLICE + kk * CHUNK, CHUNK)])
        plsc.subcore_barrier()

        cv = cvec[...]

        def p1(j, carry):
            for k in range(NV):
                sl = pl.ds(k * 16, 16)
                sv = src_v[j, sl]
                dv = dst_v[j, sl]
                av = plsc.load_gather(el_v, [sv])
                bv2 = plsc.load_gather(er_v, [dv])
                e = av + bv2
                e = jnp.where(e > 0, e, e * 0.2)
                ee = jnp.exp(e - cv)
                ee = jnp.where(j * NV + k < NV_REAL, ee, jnp.zeros_like(ee))
                ee_v[j, sl] = ee
                gix_v[j, sl] = sv * HEADS + h
            pltpu.sync_copy(ee_v.at[j], esum_s.at[dst_v.at[j]], add=True)
            return carry

        lax.fori_loop(0, NCH, p1, 0)

        def _p2_work(j, b):
            for k in range(NV):
                ee = ee_v[j, pl.ds(k * 16, 16)]
                for i2 in range(16):
                    r = k * 16 + i2
                    asp = jnp.broadcast_to(ee[i2], (16,))
                    sbuf[r, pl.ds(0, 16)] = gbuf[b, r, pl.ds(0, 16)] * asp
                    sbuf[r, pl.ds(16, 16)] = gbuf[b, r, pl.ds(16, 16)] * asp
            pltpu.sync_copy(sbuf, out_s.at[dst_v.at[j]], add=True)

        def _g(j, b, sem):
            return pltpu.make_async_copy(featv.at[gix_v.at[j]],
                                         gbuf.at[b], sem)

        pltpu.async_copy(featv.at[gix_v.at[0]], gbuf.at[0], gsem)

        def p2pair(p, carry):
            j0 = p * 2
            _g(j0 + 1, 1, gsem2).start()
            _g(j0, 0, gsem).wait()
            _p2_work(j0, 0)
            _g(j0 + 2, 0, gsem).start()
            _g(j0 + 1, 1, gsem2).wait()
            _p2_work(j0 + 1, 1)
            return carry

        lax.fori_loop(0, NCH // 2, p2pair, 0)
        _g(NCH - 1, 0, gsem).wait()
        _p2_work(NCH - 1, 0)
        plsc.subcore_barrier()

        pltpu.sync_copy(esum_s.at[pl.ds(s * SLICE, SLICE)], es_v)

        def norm_chunk(kk, carry):
            off = s * SLICE + kk * CHUNK
            pltpu.sync_copy(out_s.at[pl.ds(off, CHUNK)], gbuf.at[0])
            for k in range(NV):
                vals = es_v[pl.ds(kk * CHUNK + k * 16, 16)]
                inv = 1.0 / (vals + 1e-9)
                for i2 in range(16):
                    r = k * 16 + i2
                    isp = jnp.broadcast_to(inv[i2], (16,))
                    sbuf[r, pl.ds(0, 16)] = gbuf[0, r, pl.ds(0, 16)] * isp
                    sbuf[r, pl.ds(16, 16)] = gbuf[0, r, pl.ds(16, 16)] * isp
            pltpu.sync_copy(sbuf, out_u.at[pl.ds(off, CHUNK), h])
            return carry

        lax.fori_loop(0, SLICE // CHUNK, norm_chunk, 0)
        plsc.subcore_barrier()
        return carry

    lax.fori_loop(0, HPC, head_body, 0)


def kernel(x, edge_index, W_in, b_in, Wl, al, ar, bl, q, Wk, bk, Wv, bv,
           W1, b1, W2, b2):
    src = edge_index[0]
    dst = edge_index[1]
    srcp = jnp.pad(src.reshape(16, EPT),
                   ((0, 0), (0, EPT_P - EPT))).reshape(16, NCH, CHUNK)
    dstp = jnp.pad(dst.reshape(16, EPT),
                   ((0, 0), (0, EPT_P - EPT))).reshape(16, NCH, CHUNK)

    xp = jnp.pad(x, ((0, NPAD - N), (0, 0)))
    h = _tc_input_proj(xp, W_in, b_in.reshape(1, HID))

    karr = jnp.arange(HID)
    hsel = (karr[:, None] // DH) == jnp.arange(HEADS)[None, :]

    u = None
    for i in range(L):
        albd = jnp.where(hsel, al[i].reshape(HID, 1), 0.0).astype(_f32)
        arbd = jnp.where(hsel, ar[i].reshape(HID, 1), 0.0).astype(_f32)
        if i == 0:
            feat, elT, erT, mx = _tc_layer_first(h, Wl[i], albd, arbd)
        else:
            feat, elT, erT, mx = _tc_layer_next(
                u, bl[i - 1].reshape(1, HID), Wl[i], albd, arbd)
        cmax = jnp.maximum(0.0, mx[0] + mx[1])
        cb = jnp.broadcast_to(cmax[:, None], (HEADS, 16)).astype(_f32)
        featv = feat.reshape(NPAD * HEADS, DH)
        out_u = _sc_layer(featv, elT, erT, cb, srcp, dstp)
        u = out_u.reshape(NPAD, HID)

    return _tc_pool(u, bl[L - 1].reshape(1, HID), Wk,
                    bk.reshape(1, HID), Wv, bv.reshape(1, HID), q,
                    W1, b1.reshape(1, HID), W2, b2.reshape(1, OUT))


# speedup vs baseline: 2.8970x; 1.1267x over previous
import functools

import jax
import jax.numpy as jnp
from jax import lax
from jax.experimental import pallas as pl
from jax.experimental.pallas import tpu as pltpu
from jax.experimental.pallas import tpu_sc as plsc

N = 10000
E = 160000
IN = 256
HID = 512
HEADS = 16
DH = HID // HEADS
L = 3
OUT = 128

NPAD = 10240
SLICE = NPAD // 16
EPT = E // 16
CHUNK = 128
NCH = (EPT + CHUNK - 1) // CHUNK
EPT_P = NCH * CHUNK
NV = CHUNK // 16
NV_REAL = EPT // 16
HPC = HEADS // 2

BM = 512
GRID = NPAD // BM

_f32 = jnp.float32


def _a0_body(x_ref, w_ref, b_ref, o_ref):
    o_ref[...] = (
        jnp.dot(x_ref[...], w_ref[...], preferred_element_type=_f32)
        + b_ref[...]
    )


def _tc_input_proj(x, W, b):
    return pl.pallas_call(
        _a0_body,
        grid=(GRID,),
        in_specs=[
            pl.BlockSpec((BM, IN), lambda i: (i, 0)),
            pl.BlockSpec((IN, HID), lambda i: (0, 0)),
            pl.BlockSpec((1, HID), lambda i: (0, 0)),
        ],
        out_specs=pl.BlockSpec((BM, HID), lambda i: (i, 0)),
        out_shape=jax.ShapeDtypeStruct((NPAD, HID), _f32),
    )(x, W, b)


def _make_layer_body(pre):
    def body(*refs):
        if pre:
            (u_ref, b_ref, w_ref, al_ref, ar_ref,
             feat_ref, el_ref, er_ref, mx_ref) = refs
        else:
            (u_ref, w_ref, al_ref, ar_ref,
             feat_ref, el_ref, er_ref, mx_ref) = refs
        i = pl.program_id(0)
        a = u_ref[...]
        if pre:
            a = jnp.maximum(a + b_ref[...], 0.0)
        f = jnp.dot(a, w_ref[...], preferred_element_type=_f32)
        feat_ref[...] = f
        el = jnp.dot(f, al_ref[...], preferred_element_type=_f32)
        er = jnp.dot(f, ar_ref[...], preferred_element_type=_f32)
        el_ref[...] = el.T
        er_ref[...] = er.T
        elm = jnp.max(el, axis=0, keepdims=True)
        erm = jnp.max(er, axis=0, keepdims=True)
        new = jnp.concatenate(
            [elm, erm, jnp.full((6, HEADS), -jnp.inf, _f32)], axis=0)

        @pl.when(i == 0)
        def _():
            mx_ref[...] = new

        @pl.when(i > 0)
        def _():
            mx_ref[...] = jnp.maximum(mx_ref[...], new)

    return body


_layer_body_pre = _make_layer_body(True)
_layer_body_nopre = _make_layer_body(False)

_LAYER_OUT = (
    jax.ShapeDtypeStruct((NPAD, HID), _f32),
    jax.ShapeDtypeStruct((HEADS, NPAD), _f32),
    jax.ShapeDtypeStruct((HEADS, NPAD), _f32),
    jax.ShapeDtypeStruct((8, HEADS), _f32),
)

_LAYER_OUT_SPECS = [
    pl.BlockSpec((BM, HID), lambda i: (i, 0)),
    pl.BlockSpec((HEADS, BM), lambda i: (0, i)),
    pl.BlockSpec((HEADS, BM), lambda i: (0, i)),
    pl.BlockSpec((8, HEADS), lambda i: (0, 0)),
]


def _tc_layer_first(h, W, albd, arbd):
    return pl.pallas_call(
        _layer_body_nopre,
        grid=(GRID,),
        in_specs=[
            pl.BlockSpec((BM, HID), lambda i: (i, 0)),
            pl.BlockSpec((HID, HID), lambda i: (0, 0)),
            pl.BlockSpec((HID, HEADS), lambda i: (0, 0)),
            pl.BlockSpec((HID, HEADS), lambda i: (0, 0)),
        ],
        out_specs=_LAYER_OUT_SPECS,
        out_shape=_LAYER_OUT,
    )(h, W, albd, arbd)


def _tc_layer_next(u, b, W, albd, arbd):
    return pl.pallas_call(
        _layer_body_pre,
        grid=(GRID,),
        in_specs=[
            pl.BlockSpec((BM, HID), lambda i: (i, 0)),
            pl.BlockSpec((1, HID), lambda i: (0, 0)),
            pl.BlockSpec((HID, HID), lambda i: (0, 0)),
            pl.BlockSpec((HID, HEADS), lambda i: (0, 0)),
            pl.BlockSpec((HID, HEADS), lambda i: (0, 0)),
        ],
        out_specs=_LAYER_OUT_SPECS,
        out_shape=_LAYER_OUT,
    )(u, b, W, albd, arbd)


def _pool_body(u_ref, b_ref, wk_ref, bk_ref, wv_ref, bv_ref, q_ref,
               w1_ref, b1_ref, w2_ref, b2_ref, o_ref, acc, sm):
    i = pl.program_id(0)
    a = jnp.maximum(u_ref[...] + b_ref[...], 0.0)
    kk = jnp.dot(a, wk_ref[...], preferred_element_type=_f32) + bk_ref[...]
    vv = jnp.dot(a, wv_ref[...], preferred_element_type=_f32) + bv_ref[...]
    lg = jnp.sum(kk * q_ref[...], axis=1, keepdims=True) * _f32(HID ** -0.5)
    rows = i * BM + jax.lax.broadcasted_iota(jnp.int32, (BM, 1), 0)
    lg = jnp.where(rows < N, lg, -jnp.inf)
    bm = jnp.max(lg)

    @pl.when(i == 0)
    def _():
        sm[0] = -jnp.inf
        sm[1] = 0.0
        acc[...] = jnp.zeros((8, HID), _f32)

    prev_m = sm[0]
    prev_s = sm[1]
    prev_v = acc[0:1, :]
    new_m = jnp.maximum(prev_m, bm)
    corr = jnp.exp(prev_m - new_m)
    p = jnp.exp(lg - new_m)
    sm[0] = new_m
    sm[1] = prev_s * corr + jnp.sum(p)
    acc[0:1, :] = prev_v * corr + jnp.sum(p * vv, axis=0, keepdims=True)

    @pl.when(i == GRID - 1)
    def _():
        hg = acc[0:1, :] / sm[1]
        o1 = jnp.maximum(
            jnp.dot(hg, w1_ref[...], preferred_element_type=_f32)
            + b1_ref[...], 0.0)
        o_ref[...] = (
            jnp.dot(o1, w2_ref[...], preferred_element_type=_f32)
            + b2_ref[...]
        )


def _tc_pool(u, b, Wk, bk, Wv, bv, q, W1, b1, W2, b2):
    return pl.pallas_call(
        _pool_body,
        grid=(GRID,),
        in_specs=[
            pl.BlockSpec((BM, HID), lambda i: (i, 0)),
            pl.BlockSpec((1, HID), lambda i: (0, 0)),
            pl.BlockSpec((HID, HID), lambda i: (0, 0)),
            pl.BlockSpec((1, HID), lambda i: (0, 0)),
            pl.BlockSpec((HID, HID), lambda i: (0, 0)),
            pl.BlockSpec((1, HID), lambda i: (0, 0)),
            pl.BlockSpec((1, HID), lambda i: (0, 0)),
            pl.BlockSpec((HID, HID), lambda i: (0, 0)),
            pl.BlockSpec((1, HID), lambda i: (0, 0)),
            pl.BlockSpec((HID, OUT), lambda i: (0, 0)),
            pl.BlockSpec((1, OUT), lambda i: (0, 0)),
        ],
        out_specs=pl.BlockSpec((1, OUT), lambda i: (0, 0)),
        out_shape=jax.ShapeDtypeStruct((1, OUT), _f32),
        scratch_shapes=[
            pltpu.VMEM((8, HID), _f32),
            pltpu.SMEM((2,), _f32),
        ],
    )(u, b, Wk, bk, Wv, bv, q, W1, b1, W2, b2)


_mesh = plsc.VectorSubcoreMesh(
    core_axis_name="c", subcore_axis_name="s", num_cores=2, num_subcores=16)


@functools.partial(
    pl.kernel,
    out_type=jax.ShapeDtypeStruct((NPAD, HEADS, DH), _f32),
    mesh=_mesh,
    compiler_params=pltpu.CompilerParams(
        use_tc_tiling_on_sc=False, needs_layout_passes=False),
    scratch_types=[
        pltpu.VMEM((NPAD,), _f32),
        pltpu.VMEM((NPAD,), _f32),
        pltpu.VMEM((16,), _f32),
        pltpu.VMEM((NCH, CHUNK), jnp.int32),
        pltpu.VMEM((NCH, CHUNK), jnp.int32),
        pltpu.VMEM((NCH, CHUNK), jnp.int32),
        pltpu.VMEM((NCH, CHUNK), _f32),
        pltpu.VMEM((2, CHUNK, DH), _f32),
        pltpu.VMEM((2, CHUNK, DH), _f32),
        pltpu.VMEM((CHUNK, DH), _f32),
        pltpu.VMEM((SLICE,), _f32),
        pltpu.VMEM((SLICE,), _f32),
        pltpu.VMEM_SHARED((NPAD,), _f32),
        pltpu.VMEM_SHARED((NPAD, DH), _f32),
        pltpu.SemaphoreType.DMA,
        pltpu.SemaphoreType.DMA,
        pltpu.SemaphoreType.DMA,
        pltpu.SemaphoreType.DMA,
        pltpu.SemaphoreType.DMA,
    ],
)
def _sc_layer(featv, elT, erT, cb, srcp, dstp, out_u,
              el_v, er_v, cvec, src_v, dst_v, gix_v, ee_v, gbuf, sbuf,
              zb_v, zs_v, es_v, esum_s, out_s,
              gsem, gsem2, ssemA, ssemB, psem):
    c = lax.axis_index("c")
    s = lax.axis_index("s")
    pltpu.sync_copy(srcp.at[s], src_v)
    pltpu.sync_copy(dstp.at[s], dst_v)

    zero = jnp.zeros((16,), _f32)

    def zb_loop(r, carry):
        zb_v[r, pl.ds(0, 16)] = zero
        zb_v[r, pl.ds(16, 16)] = zero
        return carry

    lax.fori_loop(0, CHUNK, zb_loop, 0)

    def zs_loop(r, carry):
        zs_v[pl.ds(r * 16, 16)] = zero
        return carry

    lax.fori_loop(0, SLICE // 16, zs_loop, 0)

    def head_body(hl, carry):
        h = c * HPC + hl
        pltpu.sync_copy(elT.at[h], el_v)
        pltpu.sync_copy(erT.at[h], er_v)
        pltpu.sync_copy(cb.at[h], cvec)
        pltpu.sync_copy(zs_v, esum_s.at[pl.ds(s * SLICE, SLICE)])
        for kk in range(SLICE // CHUNK):
            pltpu.sync_copy(
                zb_v, out_s.at[pl.ds(s * SLICE + kk * CHUNK, CHUNK)])
        plsc.subcore_barrier()

        cv = cvec[...]

        pltpu.async_copy(zs_v.at[pl.ds(0, CHUNK)],
                         esum_s.at[dst_v.at[0]], psem, add=True)

        def p1(j, carry):
            for k in range(NV):
                sl = pl.ds(k * 16, 16)
                sv = src_v[j, sl]
                dv = dst_v[j, sl]
                av = plsc.load_gather(el_v, [sv])
                bv2 = plsc.load_gather(er_v, [dv])
                e = av + bv2
                e = jnp.where(e > 0, e, e * 0.2)
                ee = jnp.exp(e - cv)
                ee = jnp.where(j * NV + k < NV_REAL, ee, jnp.zeros_like(ee))
                ee_v[j, sl] = ee
                gix_v[j, sl] = sv * HEADS + h
            pltpu.async_copy(ee_v.at[j], esum_s.at[dst_v.at[j]], psem,
                             add=True)
            pltpu.make_async_copy(
                ee_v.at[jnp.maximum(j - 1, 0)],
                esum_s.at[dst_v.at[jnp.maximum(j - 1, 0)]], psem).wait()
            return carry

        lax.fori_loop(0, NCH, p1, 0)

        def _p2_work(j, b):
            for k in range(NV):
                ee = ee_v[j, pl.ds(k * 16, 16)]
                for i2 in range(16):
                    r = k * 16 + i2
                    asp = jnp.broadcast_to(ee[i2], (16,))
                    sbuf[b, r, pl.ds(0, 16)] = gbuf[b, r, pl.ds(0, 16)] * asp
                    sbuf[b, r, pl.ds(16, 16)] = gbuf[b, r, pl.ds(16, 16)] * asp

        def _g(j, b, sem):
            return pltpu.make_async_copy(featv.at[gix_v.at[j]],
                                         gbuf.at[b], sem)

        def _sc(j, b, sem):
            return pltpu.make_async_copy(sbuf.at[b], out_s.at[dst_v.at[j]],
                                         sem)

        pltpu.async_copy(featv.at[gix_v.at[0]], gbuf.at[0], gsem)
        pltpu.async_copy(zb_v, out_s.at[dst_v.at[0]], ssemA, add=True)
        pltpu.async_copy(zb_v, out_s.at[dst_v.at[0]], ssemB, add=True)

        def p2pair(p, carry):
            j0 = p * 2
            jp0 = jnp.maximum(j0 - 2, 0)
            _g(j0 + 1, 1, gsem2).start()
            _g(j0, 0, gsem).wait()
            _sc(jp0, 0, ssemA).wait()
            _p2_work(j0, 0)
            pltpu.async_copy(sbuf.at[0], out_s.at[dst_v.at[j0]],
                             ssemA, add=True)
            _g(j0 + 2, 0, gsem).start()
            _g(j0 + 1, 1, gsem2).wait()
            _sc(jp0, 1, ssemB).wait()
            _p2_work(j0 + 1, 1)
            pltpu.async_copy(sbuf.at[1], out_s.at[dst_v.at[j0 + 1]],
                             ssemB, add=True)
            return carry

        lax.fori_loop(0, NCH // 2, p2pair, 0)
        _g(NCH - 1, 0, gsem).wait()
        _sc(NCH - 3, 0, ssemA).wait()
        _p2_work(NCH - 1, 0)
        pltpu.async_copy(sbuf.at[0], out_s.at[dst_v.at[NCH - 1]],
                         ssemA, add=True)
        _sc(NCH - 1, 0, ssemA).wait()
        _sc(NCH - 2, 1, ssemB).wait()
        pltpu.make_async_copy(ee_v.at[NCH - 1],
                              esum_s.at[dst_v.at[NCH - 1]], psem).wait()
        plsc.subcore_barrier()

        pltpu.sync_copy(esum_s.at[pl.ds(s * SLICE, SLICE)], es_v)

        def norm_chunk(kk, carry):
            off = s * SLICE + kk * CHUNK
            pltpu.sync_copy(out_s.at[pl.ds(off, CHUNK)], gbuf.at[0])
            for k in range(NV):
                vals = es_v[pl.ds(kk * CHUNK + k * 16, 16)]
                inv = 1.0 / (vals + 1e-9)
                for i2 in range(16):
                    r = k * 16 + i2
                    isp = jnp.broadcast_to(inv[i2], (16,))
                    sbuf[0, r, pl.ds(0, 16)] = gbuf[0, r, pl.ds(0, 16)] * isp
                    sbuf[0, r, pl.ds(16, 16)] = gbuf[0, r, pl.ds(16, 16)] * isp
            pltpu.sync_copy(sbuf.at[0], out_u.at[pl.ds(off, CHUNK), h])
            return carry

        lax.fori_loop(0, SLICE // CHUNK, norm_chunk, 0)
        plsc.subcore_barrier()
        return carry

    lax.fori_loop(0, HPC, head_body, 0)


def kernel(x, edge_index, W_in, b_in, Wl, al, ar, bl, q, Wk, bk, Wv, bv,
           W1, b1, W2, b2):
    src = edge_index[0]
    dst = edge_index[1]
    srcp = jnp.pad(src.reshape(16, EPT),
                   ((0, 0), (0, EPT_P - EPT))).reshape(16, NCH, CHUNK)
    dstp = jnp.pad(dst.reshape(16, EPT),
                   ((0, 0), (0, EPT_P - EPT))).reshape(16, NCH, CHUNK)

    xp = jnp.pad(x, ((0, NPAD - N), (0, 0)))
    h = _tc_input_proj(xp, W_in, b_in.reshape(1, HID))

    karr = jnp.arange(HID)
    hsel = (karr[:, None] // DH) == jnp.arange(HEADS)[None, :]

    u = None
    for i in range(L):
        albd = jnp.where(hsel, al[i].reshape(HID, 1), 0.0).astype(_f32)
        arbd = jnp.where(hsel, ar[i].reshape(HID, 1), 0.0).astype(_f32)
        if i == 0:
            feat, elT, erT, mx = _tc_layer_first(h, Wl[i], albd, arbd)
        else:
            feat, elT, erT, mx = _tc_layer_next(
                u, bl[i - 1].reshape(1, HID), Wl[i], albd, arbd)
        cmax = jnp.maximum(0.0, mx[0] + mx[1])
        cb = jnp.broadcast_to(cmax[:, None], (HEADS, 16)).astype(_f32)
        featv = feat.reshape(NPAD * HEADS, DH)
        out_u = _sc_layer(featv, elT, erT, cb, srcp, dstp)
        u = out_u.reshape(NPAD, HID)

    return _tc_pool(u, bl[L - 1].reshape(1, HID), Wk,
                    bk.reshape(1, HID), Wv, bv.reshape(1, HID), q,
                    W1, b1.reshape(1, HID), W2, b2.reshape(1, OUT))
